# Initial kernel scaffold; baseline (speedup 1.0000x reference)
import jax
import jax.numpy as jnp
from jax.experimental import pallas as pl


def kernel(user_id, hist_item, item_id, neg_items, user_table, item_table, convert_w, linear_w, capsule_weight_init):
    raise NotImplementedError("write your pallas kernel here")


# speedup vs baseline: 2.5835x; 2.5835x over previous
import functools

import jax
import jax.numpy as jnp
from jax import lax
from jax.experimental import pallas as pl
from jax.experimental.pallas import tpu as pltpu
from jax.experimental.pallas import tpu_sc as plsc

B = 4096
S = 50
NNEG = 10
D = 32
K = 4
ROUTING = 3

NC = 2
NS = 16
NW = NC * NS
CHUNK = 128

NI = B * S + B + B * NNEG
PER_W = NI // NW
NCHUNK = PER_W // CHUNK
UPER_W = B // NW

assert PER_W % CHUNK == 0 and UPER_W == CHUNK


def _sc_gather_body(item_hbm, user_hbm, iidx_hbm, uidx_hbm, out_i, out_u,
                    idx_v, uidx_v, buf, ubuf, sem, usem):
    cid = lax.axis_index("c")
    sid = lax.axis_index("s")
    wid = sid * NC + cid
    pltpu.sync_copy(iidx_hbm.at[wid], idx_v)
    pltpu.sync_copy(uidx_hbm.at[wid], uidx_v)
    base = wid * PER_W

    pltpu.async_copy(user_hbm.at[uidx_v], ubuf, usem).wait()
    pltpu.sync_copy(ubuf, out_u.at[pl.ds(wid * UPER_W, UPER_W)])

    @pl.loop(0, NCHUNK)
    def _step(j):
        pltpu.async_copy(item_hbm.at[idx_v.at[j]], buf, sem).wait()
        pltpu.sync_copy(buf, out_i.at[pl.ds(base + j * CHUNK, CHUNK)])


@jax.jit
def _sc_gather(item_table, user_table, iidx3, uidx2):
    mesh = plsc.VectorSubcoreMesh(core_axis_name="c", subcore_axis_name="s")
    return pl.kernel(
        _sc_gather_body,
        out_type=(
            jax.ShapeDtypeStruct((NI, D), jnp.float32),
            jax.ShapeDtypeStruct((B, D), jnp.float32),
        ),
        mesh=mesh,
        compiler_params=pltpu.CompilerParams(use_tc_tiling_on_sc=False),
        scratch_types=[
            pltpu.VMEM((NCHUNK, CHUNK), jnp.int32),
            pltpu.VMEM((CHUNK,), jnp.int32),
            pltpu.VMEM((CHUNK, D), jnp.float32),
            pltpu.VMEM((CHUNK, D), jnp.float32),
            pltpu.SemaphoreType.DMA,
            pltpu.SemaphoreType.DMA,
        ],
    )(item_table, user_table, iidx3, uidx2)


def _l2n(x):
    n = jnp.sqrt(jnp.sum(x * x, axis=-1, keepdims=True))
    return x / jnp.maximum(n, 1e-12)


def _tc_body(hist_rows, hist_item, urows, prows, nrows, cw0, lin_w, conv_w, y):
    bb = hist_rows.shape[0]
    hat = jnp.dot(hist_rows[...].reshape(bb * S, D), lin_w[...],
                  preferred_element_type=jnp.float32)
    hat3 = hat.reshape(bb, S, D)
    mask = hist_item[...] > 0
    cw = cw0[...]

    interests = None
    for it in range(ROUTING):
        m = jnp.max(cw, axis=-1, keepdims=True)
        e = jnp.exp(cw - m)
        w = e / jnp.sum(e, axis=-1, keepdims=True)
        w = jnp.where(mask[:, None, :], w, 0.0)
        ints = []
        for k in range(K):
            ik = jnp.sum(w[:, k, :][:, :, None] * hat3, axis=1)
            cap = jnp.sum(ik * ik, axis=-1, keepdims=True)
            scal = cap / (1.0 + cap) / jnp.sqrt(cap + 1e-9)
            ints.append(scal * ik)
        if it < ROUTING - 1:
            deltas = [jnp.sum(hat3 * ints[k][:, None, :], axis=2)[:, None, :]
                      for k in range(K)]
            cw = cw + jnp.concatenate(deltas, axis=1)
        else:
            interests = ints

    ue = urows[...]
    posn = _l2n(prows[...])
    neg = nrows[...]
    item_sum = posn + jnp.sum(_l2n(neg), axis=1)

    best = None
    bestd = None
    for k in range(K):
        iu = jnp.concatenate([ue, interests[k]], axis=1)
        uek = _l2n(jnp.dot(iu, conv_w[...], preferred_element_type=jnp.float32))
        dk = jnp.sum(uek * posn, axis=1, keepdims=True)
        if k == 0:
            best, bestd = uek, dk
        else:
            take = dk > bestd
            best = jnp.where(take, uek, best)
            bestd = jnp.where(take, dk, bestd)
    y[...] = best * item_sum


@functools.partial(jax.jit, static_argnames=("bb",))
def _tc_compute(hist_rows, hist_item, urows, prows, nrows, cw0, lin_w, conv_w,
                bb=128):
    grid = (B // bb,)
    return pl.pallas_call(
        _tc_body,
        grid=grid,
        in_specs=[
            pl.BlockSpec((bb, S, D), lambda i: (i, 0, 0)),
            pl.BlockSpec((bb, S), lambda i: (i, 0)),
            pl.BlockSpec((bb, D), lambda i: (i, 0)),
            pl.BlockSpec((bb, D), lambda i: (i, 0)),
            pl.BlockSpec((bb, NNEG, D), lambda i: (i, 0, 0)),
            pl.BlockSpec((bb, K, S), lambda i: (i, 0, 0)),
            pl.BlockSpec((D, D), lambda i: (0, 0)),
            pl.BlockSpec((2 * D, D), lambda i: (0, 0)),
        ],
        out_specs=pl.BlockSpec((bb, D), lambda i: (i, 0)),
        out_shape=jax.ShapeDtypeStruct((B, D), jnp.float32),
    )(hist_rows, hist_item, urows, prows, nrows, cw0, lin_w, conv_w)


def kernel(user_id, hist_item, item_id, neg_items, user_table, item_table,
           convert_w, linear_w, capsule_weight_init):
    item_idx = jnp.concatenate(
        [hist_item.reshape(-1), item_id, neg_items.reshape(-1)]
    ).astype(jnp.int32)
    iidx3 = item_idx.reshape(NW, NCHUNK, CHUNK)
    uidx2 = user_id.astype(jnp.int32).reshape(NW, CHUNK)

    rows, urows = _sc_gather(item_table, user_table, iidx3, uidx2)
    hist_rows = rows[: B * S].reshape(B, S, D)
    prows = rows[B * S: B * S + B]
    nrows = rows[B * S + B:].reshape(B, NNEG, D)

    return _tc_compute(hist_rows, hist_item, urows, prows, nrows,
                       capsule_weight_init, linear_w, convert_w)


# speedup vs baseline: 2.8525x; 1.1041x over previous
import functools

import jax
import jax.numpy as jnp
from jax import lax
from jax.experimental import pallas as pl
from jax.experimental.pallas import tpu as pltpu
from jax.experimental.pallas import tpu_sc as plsc

B = 4096
S = 50
NNEG = 10
D = 32
K = 4
ROUTING = 3

NC = 2
NS = 16
NW = NC * NS
CHUNK = 128

HCH = (B * S) // (NW * CHUNK)
NCH = (B * NNEG) // (NW * CHUNK)

assert HCH * NW * CHUNK == B * S and NCH * NW * CHUNK == B * NNEG
assert B // NW == CHUNK


def _sc_gather_body(item_hbm, user_hbm, hidx_hbm, nidx_hbm, pidx_hbm, uidx_hbm,
                    out_h, out_n, out_p, out_u,
                    hidx_v, nidx_v, pidx_v, uidx_v, buf, sbuf, sem, ssem):
    cid = lax.axis_index("c")
    sid = lax.axis_index("s")
    wid = sid * NC + cid
    pltpu.sync_copy(hidx_hbm.at[wid], hidx_v)
    pltpu.sync_copy(nidx_hbm.at[wid], nidx_v)
    pltpu.sync_copy(pidx_hbm.at[wid], pidx_v)
    pltpu.sync_copy(uidx_hbm.at[wid], uidx_v)

    pltpu.async_copy(item_hbm.at[pidx_v], sbuf, ssem).wait()
    pltpu.sync_copy(sbuf, out_p.at[pl.ds(wid * CHUNK, CHUNK)])
    pltpu.async_copy(user_hbm.at[uidx_v], sbuf, ssem).wait()
    pltpu.sync_copy(sbuf, out_u.at[pl.ds(wid * CHUNK, CHUNK)])

    nbase = wid * NCH * CHUNK

    @pl.loop(0, NCH)
    def _nstep(j):
        pltpu.async_copy(item_hbm.at[nidx_v.at[j]], buf, sem).wait()
        pltpu.sync_copy(buf, out_n.at[pl.ds(nbase + j * CHUNK, CHUNK)])

    hbase = wid * HCH * CHUNK

    @pl.loop(0, HCH)
    def _hstep(j):
        pltpu.async_copy(item_hbm.at[hidx_v.at[j]], buf, sem).wait()
        pltpu.sync_copy(buf, out_h.at[pl.ds(hbase + j * CHUNK, CHUNK)])


@jax.jit
def _sc_gather(item_table, user_table, hidx3, nidx3, pidx2, uidx2):
    mesh = plsc.VectorSubcoreMesh(core_axis_name="c", subcore_axis_name="s")
    return pl.kernel(
        _sc_gather_body,
        out_type=(
            jax.ShapeDtypeStruct((B * S, D), jnp.float32),
            jax.ShapeDtypeStruct((B * NNEG, D), jnp.float32),
            jax.ShapeDtypeStruct((B, D), jnp.float32),
            jax.ShapeDtypeStruct((B, D), jnp.float32),
        ),
        mesh=mesh,
        compiler_params=pltpu.CompilerParams(use_tc_tiling_on_sc=False),
        scratch_types=[
            pltpu.VMEM((HCH, CHUNK), jnp.int32),
            pltpu.VMEM((NCH, CHUNK), jnp.int32),
            pltpu.VMEM((CHUNK,), jnp.int32),
            pltpu.VMEM((CHUNK,), jnp.int32),
            pltpu.VMEM((CHUNK, D), jnp.float32),
            pltpu.VMEM((CHUNK, D), jnp.float32),
            pltpu.SemaphoreType.DMA,
            pltpu.SemaphoreType.DMA,
        ],
    )(item_table, user_table, hidx3, nidx3, pidx2, uidx2)


def _l2n(x):
    n = jnp.sqrt(jnp.sum(x * x, axis=-1, keepdims=True))
    return x / jnp.maximum(n, 1e-12)


def _tc_body(hist_rows, hist_item, urows, prows, nrows, cw0, lin_w, conv_w, y):
    bb = hist_rows.shape[0]
    hat = jnp.dot(hist_rows[...].reshape(bb * S, D), lin_w[...],
                  preferred_element_type=jnp.float32)
    hat3 = hat.reshape(bb, S, D)
    mask = hist_item[...] > 0
    cw = cw0[...]

    interests = None
    for it in range(ROUTING):
        m = jnp.max(cw, axis=-1, keepdims=True)
        e = jnp.exp(cw - m)
        w = e / jnp.sum(e, axis=-1, keepdims=True)
        w = jnp.where(mask[:, None, :], w, 0.0)
        ints = []
        for k in range(K):
            ik = jnp.sum(w[:, k, :][:, :, None] * hat3, axis=1)
            cap = jnp.sum(ik * ik, axis=-1, keepdims=True)
            scal = cap / (1.0 + cap) / jnp.sqrt(cap + 1e-9)
            ints.append(scal * ik)
        if it < ROUTING - 1:
            deltas = [jnp.sum(hat3 * ints[k][:, None, :], axis=2)[:, None, :]
                      for k in range(K)]
            cw = cw + jnp.concatenate(deltas, axis=1)
        else:
            interests = ints

    ue = urows[...]
    posn = _l2n(prows[...])
    neg = nrows[...]
    item_sum = posn + jnp.sum(_l2n(neg), axis=1)

    best = None
    bestd = None
    for k in range(K):
        iu = jnp.concatenate([ue, interests[k]], axis=1)
        uek = _l2n(jnp.dot(iu, conv_w[...], preferred_element_type=jnp.float32))
        dk = jnp.sum(uek * posn, axis=1, keepdims=True)
        if k == 0:
            best, bestd = uek, dk
        else:
            take = dk > bestd
            best = jnp.where(take, uek, best)
            bestd = jnp.where(take, dk, bestd)
    y[...] = best * item_sum


@functools.partial(jax.jit, static_argnames=("bb",))
def _tc_compute(hist_rows, hist_item, urows, prows, nrows, cw0, lin_w, conv_w,
                bb=128):
    grid = (B // bb,)
    return pl.pallas_call(
        _tc_body,
        grid=grid,
        in_specs=[
            pl.BlockSpec((bb, S, D), lambda i: (i, 0, 0)),
            pl.BlockSpec((bb, S), lambda i: (i, 0)),
            pl.BlockSpec((bb, D), lambda i: (i, 0)),
            pl.BlockSpec((bb, D), lambda i: (i, 0)),
            pl.BlockSpec((bb, NNEG, D), lambda i: (i, 0, 0)),
            pl.BlockSpec((bb, K, S), lambda i: (i, 0, 0)),
            pl.BlockSpec((D, D), lambda i: (0, 0)),
            pl.BlockSpec((2 * D, D), lambda i: (0, 0)),
        ],
        out_specs=pl.BlockSpec((bb, D), lambda i: (i, 0)),
        out_shape=jax.ShapeDtypeStruct((B, D), jnp.float32),
    )(hist_rows, hist_item, urows, prows, nrows, cw0, lin_w, conv_w)


def kernel(user_id, hist_item, item_id, neg_items, user_table, item_table,
           convert_w, linear_w, capsule_weight_init):
    hidx3 = hist_item.astype(jnp.int32).reshape(NW, HCH, CHUNK)
    nidx3 = neg_items.astype(jnp.int32).reshape(NW, NCH, CHUNK)
    pidx2 = item_id.astype(jnp.int32).reshape(NW, CHUNK)
    uidx2 = user_id.astype(jnp.int32).reshape(NW, CHUNK)

    hrows, xnrows, prows, urows = _sc_gather(
        item_table, user_table, hidx3, nidx3, pidx2, uidx2)
    hist_rows = hrows.reshape(B, S, D)
    nrows = xnrows.reshape(B, NNEG, D)

    return _tc_compute(hist_rows, hist_item, urows, prows, nrows,
                       capsule_weight_init, linear_w, convert_w)


# speedup vs baseline: 2.8921x; 1.0139x over previous
import functools

import jax
import jax.numpy as jnp
from jax import lax
from jax.experimental import pallas as pl
from jax.experimental.pallas import tpu as pltpu
from jax.experimental.pallas import tpu_sc as plsc

B = 4096
S = 50
NNEG = 10
D = 32
K = 4
ROUTING = 3

NC = 2
NS = 16
NW = NC * NS
CHUNK = 128

HCH = (B * S) // (NW * CHUNK)
NCH = (B * NNEG) // (NW * CHUNK)

assert HCH * NW * CHUNK == B * S and NCH * NW * CHUNK == B * NNEG
assert B // NW == CHUNK


def _sc_gather_body(item_hbm, user_hbm, hidx_hbm, nidx_hbm, pidx_hbm, uidx_hbm,
                    out_h, out_n, out_p, out_u,
                    hidx_v, nidx_v, pidx_v, uidx_v, buf, sbuf, sem, ssem):
    cid = lax.axis_index("c")
    sid = lax.axis_index("s")
    wid = sid * NC + cid
    pltpu.sync_copy(hidx_hbm.at[wid], hidx_v)
    pltpu.sync_copy(nidx_hbm.at[wid], nidx_v)
    pltpu.sync_copy(pidx_hbm.at[wid], pidx_v)
    pltpu.sync_copy(uidx_hbm.at[wid], uidx_v)

    pltpu.async_copy(item_hbm.at[pidx_v], sbuf, ssem).wait()
    pltpu.sync_copy(sbuf, out_p.at[pl.ds(wid * CHUNK, CHUNK)])
    pltpu.async_copy(user_hbm.at[uidx_v], sbuf, ssem).wait()
    pltpu.sync_copy(sbuf, out_u.at[pl.ds(wid * CHUNK, CHUNK)])

    nbase = wid * NCH * CHUNK

    @pl.loop(0, NCH)
    def _nstep(j):
        pltpu.async_copy(item_hbm.at[nidx_v.at[j]], buf, sem).wait()
        pltpu.sync_copy(buf, out_n.at[pl.ds(nbase + j * CHUNK, CHUNK)])

    hbase = wid * HCH * CHUNK

    @pl.loop(0, HCH)
    def _hstep(j):
        pltpu.async_copy(item_hbm.at[hidx_v.at[j]], buf, sem).wait()
        pltpu.sync_copy(buf, out_h.at[pl.ds(hbase + j * CHUNK, CHUNK)])


@jax.jit
def _sc_gather(item_table, user_table, hidx3, nidx3, pidx2, uidx2):
    mesh = plsc.VectorSubcoreMesh(core_axis_name="c", subcore_axis_name="s")
    return pl.kernel(
        _sc_gather_body,
        out_type=(
            jax.ShapeDtypeStruct((B * S, D), jnp.float32),
            jax.ShapeDtypeStruct((B * NNEG, D), jnp.float32),
            jax.ShapeDtypeStruct((B, D), jnp.float32),
            jax.ShapeDtypeStruct((B, D), jnp.float32),
        ),
        mesh=mesh,
        compiler_params=pltpu.CompilerParams(use_tc_tiling_on_sc=False),
        scratch_types=[
            pltpu.VMEM((HCH, CHUNK), jnp.int32),
            pltpu.VMEM((NCH, CHUNK), jnp.int32),
            pltpu.VMEM((CHUNK,), jnp.int32),
            pltpu.VMEM((CHUNK,), jnp.int32),
            pltpu.VMEM((CHUNK, D), jnp.float32),
            pltpu.VMEM((CHUNK, D), jnp.float32),
            pltpu.SemaphoreType.DMA,
            pltpu.SemaphoreType.DMA,
        ],
    )(item_table, user_table, hidx3, nidx3, pidx2, uidx2)


def _l2n(x):
    n = jnp.sqrt(jnp.sum(x * x, axis=-1, keepdims=True))
    return x / jnp.maximum(n, 1e-12)


def _tc_body(hist_rows, hist_item, urows, prows, nrows, cw0, lin_w, conv_w, y):
    bb = hist_item.shape[0]
    hat = jnp.dot(hist_rows[...], lin_w[...],
                  preferred_element_type=jnp.float32)
    hat3 = hat.reshape(bb, S, D)
    mask = hist_item[...] > 0
    cw = cw0[...]

    interests = None
    for it in range(ROUTING):
        m = jnp.max(cw, axis=-1, keepdims=True)
        e = jnp.exp(cw - m)
        w = e / jnp.sum(e, axis=-1, keepdims=True)
        w = jnp.where(mask[:, None, :], w, 0.0)
        ints = []
        for k in range(K):
            ik = jnp.sum(w[:, k, :][:, :, None] * hat3, axis=1)
            cap = jnp.sum(ik * ik, axis=-1, keepdims=True)
            scal = cap / (1.0 + cap) / jnp.sqrt(cap + 1e-9)
            ints.append(scal * ik)
        if it < ROUTING - 1:
            deltas = [jnp.sum(hat3 * ints[k][:, None, :], axis=2)[:, None, :]
                      for k in range(K)]
            cw = cw + jnp.concatenate(deltas, axis=1)
        else:
            interests = ints

    ue = urows[...]
    posn = _l2n(prows[...])
    neg = nrows[...].reshape(bb, NNEG, D)
    item_sum = posn + jnp.sum(_l2n(neg), axis=1)

    best = None
    bestd = None
    for k in range(K):
        iu = jnp.concatenate([ue, interests[k]], axis=1)
        uek = _l2n(jnp.dot(iu, conv_w[...], preferred_element_type=jnp.float32))
        dk = jnp.sum(uek * posn, axis=1, keepdims=True)
        if k == 0:
            best, bestd = uek, dk
        else:
            take = dk > bestd
            best = jnp.where(take, uek, best)
            bestd = jnp.where(take, dk, bestd)
    y[...] = best * item_sum


@functools.partial(jax.jit, static_argnames=("bb",))
def _tc_compute(hist_rows, hist_item, urows, prows, nrows, cw0, lin_w, conv_w,
                bb=128):
    grid = (B // bb,)
    return pl.pallas_call(
        _tc_body,
        grid=grid,
        in_specs=[
            pl.BlockSpec((bb * S, D), lambda i: (i, 0)),
            pl.BlockSpec((bb, S), lambda i: (i, 0)),
            pl.BlockSpec((bb, D), lambda i: (i, 0)),
            pl.BlockSpec((bb, D), lambda i: (i, 0)),
            pl.BlockSpec((bb * NNEG, D), lambda i: (i, 0)),
            pl.BlockSpec((bb, K, S), lambda i: (i, 0, 0)),
            pl.BlockSpec((D, D), lambda i: (0, 0)),
            pl.BlockSpec((2 * D, D), lambda i: (0, 0)),
        ],
        out_specs=pl.BlockSpec((bb, D), lambda i: (i, 0)),
        out_shape=jax.ShapeDtypeStruct((B, D), jnp.float32),
    )(hist_rows, hist_item, urows, prows, nrows, cw0, lin_w, conv_w)


def kernel(user_id, hist_item, item_id, neg_items, user_table, item_table,
           convert_w, linear_w, capsule_weight_init):
    hidx3 = hist_item.astype(jnp.int32).reshape(NW, HCH, CHUNK)
    nidx3 = neg_items.astype(jnp.int32).reshape(NW, NCH, CHUNK)
    pidx2 = item_id.astype(jnp.int32).reshape(NW, CHUNK)
    uidx2 = user_id.astype(jnp.int32).reshape(NW, CHUNK)

    hrows, xnrows, prows, urows = _sc_gather(
        item_table, user_table, hidx3, nidx3, pidx2, uidx2)

    return _tc_compute(hrows, hist_item, urows, prows, xnrows,
                       capsule_weight_init, linear_w, convert_w)
